# Initial kernel scaffold; baseline (speedup 1.0000x reference)
import jax
import jax.numpy as jnp
from jax.experimental import pallas as pl


def kernel(feat, edge_index, e_feat, fc, edge_emb, fc_e_w, attn_l, attn_r, attn_e):
    raise NotImplementedError("write your pallas kernel here")


# speedup vs baseline: 24.1115x; 24.1115x over previous
import functools

import jax
import jax.numpy as jnp
from jax import lax
from jax.experimental import pallas as pl
from jax.experimental.pallas import tpu as pltpu
from jax.experimental.pallas import tpu_sc as plsc

N = 50000
E = 800000
H = 4
IN_F = 64
OUT_F = 8
NTYPE = 2
EF = 16
NET = 5

ROWA = 32
ROWB = 16
C = 128
NSUB = 16
EPAD = ((E + C * NSUB - 1) // (C * NSUB)) * (C * NSUB)
NCHUNK = EPAD // C
NCPT = NCHUNK // NSUB
ACCR = N + 16
ACCW = 6272
DR = 128
NDRAIN = N // DR
DRAIN_REM = N - NDRAIN * DR


def _tables_body(feat_ref, fc0_ref, fc1_ref, al_ref, ar_ref, ta_ref, tb_ref):
    f = feat_ref[...]
    fs0 = jnp.dot(f[:, :IN_F], fc0_ref[...],
                  preferred_element_type=jnp.float32)
    fs1 = jnp.dot(f[:, IN_F:], fc1_ref[...],
                  preferred_element_type=jnp.float32)
    al = al_ref[...]
    ar = ar_ref[...]
    bn = f.shape[0]
    fsrc = []
    els = []
    ers = []
    for h in range(H):
        fh = jnp.concatenate([fs0[:, 8 * h:8 * h + 8],
                              fs1[:, 8 * h:8 * h + 8]], axis=1)
        fsrc.append(fh)
        els.append(jnp.sum(fh * al[h][None, :], axis=1, keepdims=True))
        ers.append(jnp.sum(fh * ar[h][None, :], axis=1, keepdims=True))
    zeros8 = jnp.zeros((bn, 8), jnp.float32)
    for c in range(2):
        ta_ref[c] = jnp.concatenate(
            [fsrc[2 * c], fsrc[2 * c + 1]], axis=1)
    tb_ref[...] = jnp.concatenate(els + ers + [zeros8], axis=1)


def _node_tables(feat, fc0, fc1, al8, ar8):
    BN = 2000
    grid = (N // BN,)
    return pl.pallas_call(
        _tables_body,
        grid=grid,
        in_specs=[
            pl.BlockSpec((BN, NTYPE * IN_F), lambda i: (i, 0)),
            pl.BlockSpec((IN_F, OUT_F * H), lambda i: (0, 0)),
            pl.BlockSpec((IN_F, OUT_F * H), lambda i: (0, 0)),
            pl.BlockSpec((8, 16), lambda i: (0, 0)),
            pl.BlockSpec((8, 16), lambda i: (0, 0)),
        ],
        out_specs=[
            pl.BlockSpec((2, BN, ROWA), lambda i: (0, i, 0)),
            pl.BlockSpec((BN, ROWB), lambda i: (i, 0)),
        ],
        out_shape=[
            jax.ShapeDtypeStruct((2, N, ROWA), jnp.float32),
            jax.ShapeDtypeStruct((N, ROWB), jnp.float32),
        ],
    )(feat, fc0, fc1, al8, ar8)


def _ee_body(emb_ref, w_ref, ae_ref, out_ref):
    eh = jnp.dot(emb_ref[...], w_ref[...],
                 preferred_element_type=jnp.float32)
    ae = ae_ref[...]
    cols = []
    for h in range(H):
        cols.append(jnp.sum(eh[:, 16 * h:16 * h + 16] * ae[h][None, :],
                            axis=1, keepdims=True))
    out_ref[...] = jnp.concatenate(cols, axis=1)


def _ee_table(emb8, fc_e_w, ae8):
    return pl.pallas_call(
        _ee_body,
        out_shape=jax.ShapeDtypeStruct((8, H), jnp.float32),
    )(emb8, fc_e_w, ae8)


def _full16(val):
    return jnp.full((16,), val, dtype=jnp.int32)


def _sc_edge(ta2, tb, idx, ee):
    mesh = plsc.VectorSubcoreMesh(core_axis_name="c", subcore_axis_name="s",
                                  num_cores=2, num_subcores=NSUB)

    @functools.partial(
        pl.kernel,
        mesh=mesh,
        compiler_params=pltpu.CompilerParams(
            needs_layout_passes=False, use_tc_tiling_on_sc=False),
        out_type=[
            jax.ShapeDtypeStruct((2 * N, ROWA), jnp.float32),
            jax.ShapeDtypeStruct((2 * ACCW, ROWB), jnp.float32),
        ],
        scratch_types=[
            pltpu.VMEM((C, 8), jnp.int32),
            pltpu.VMEM((C,), jnp.int32),
            pltpu.VMEM((C,), jnp.int32),
            pltpu.VMEM((C,), jnp.int32),
            pltpu.VMEM((C,), jnp.int32),
            pltpu.VMEM((C,), jnp.int32),
            pltpu.VMEM((C, ROWA), jnp.float32),
            pltpu.VMEM((C, ROWB), jnp.float32),
            pltpu.VMEM((C, ROWB), jnp.float32),
            pltpu.VMEM((C, ROWA), jnp.float32),
            pltpu.VMEM((C, ROWB), jnp.float32),
            pltpu.VMEM((8, H), jnp.float32),
            pltpu.VMEM_SHARED((ACCR, ROWA), jnp.float32),
            pltpu.VMEM_SHARED((ACCW, ROWB), jnp.float32),
            pltpu.SemaphoreType.DMA,
        ],
    )
    def k(ta_hbm, tb_hbm, idx_hbm, ee_hbm, outf_hbm, outw_hbm,
          idx_v, sadj_v, src_v, dstc_v, dsts_v, dstw_v,
          sr_v, lb_v, rb_v, pl_v, plw_v, ee_v, accf_sh, accw_sh, sem1):
        c = lax.axis_index("c")
        s = lax.axis_index("s")
        ids16 = lax.iota(jnp.int32, 16)
        zero16 = jnp.zeros((16,), jnp.float32)
        cN16 = _full16(c * N)
        c2 = 2 * c

        for j in range(C // 16):
            eidx = ids16 + j * 16
            for col in range(ROWA):
                plsc.store_scatter(pl_v, [eidx, _full16(col)], zero16)
            for col in range(ROWB):
                plsc.store_scatter(plw_v, [eidx, _full16(col)], zero16)

        pltpu.sync_copy(ee_hbm, ee_v)

        def zacc(i, _):
            t = s + i * NSUB

            @pl.when(t < NDRAIN)
            def _():
                pltpu.sync_copy(pl_v, accf_sh.at[pl.ds(t * DR, DR)])

            @pl.when(t < ACCW // DR)
            def _():
                pltpu.sync_copy(plw_v, accw_sh.at[pl.ds(t * DR, DR)])

            return 0

        nzmax = (NDRAIN + NSUB - 1) // NSUB
        lax.fori_loop(0, nzmax, zacc, 0)

        @pl.when(s == NDRAIN % NSUB)
        def _():
            pltpu.sync_copy(pl_v.at[pl.ds(0, DRAIN_REM + 16)],
                            accf_sh.at[pl.ds(NDRAIN * DR, DRAIN_REM + 16)])

        plsc.subcore_barrier()

        def chunk(i, _):
            base = (s + i * NSUB) * C
            pltpu.sync_copy(idx_hbm.at[pl.ds(base, C)], idx_v)
            for j in range(C // 16):
                eidx = ids16 + j * 16
                g0 = plsc.load_gather(idx_v, [eidx, _full16(0)])
                plsc.store_scatter(src_v, [eidx], g0)
                plsc.store_scatter(sadj_v, [eidx], g0 + cN16)
                g1 = plsc.load_gather(idx_v, [eidx, _full16(1)])
                plsc.store_scatter(dstc_v, [eidx], g1)
                g3 = plsc.load_gather(idx_v, [eidx, _full16(3)])
                plsc.store_scatter(dsts_v, [eidx], g3)
                g4 = plsc.load_gather(idx_v, [eidx, _full16(4)])
                plsc.store_scatter(dstw_v, [eidx], g4)
            pltpu.async_copy(ta_hbm.at[sadj_v], sr_v, sem1).wait()
            pltpu.async_copy(tb_hbm.at[src_v], lb_v, sem1).wait()
            pltpu.async_copy(tb_hbm.at[dstc_v], rb_v, sem1).wait()
            for j in range(C // 16):
                eidx = ids16 + j * 16
                el0 = plsc.load_gather(lb_v, [eidx, _full16(c2)])
                el1 = plsc.load_gather(lb_v, [eidx, _full16(c2 + 1)])
                er0 = plsc.load_gather(rb_v, [eidx, _full16(4 + c2)])
                er1 = plsc.load_gather(rb_v, [eidx, _full16(4 + c2 + 1)])
                efj = plsc.load_gather(idx_v, [eidx, _full16(2)])
                ee0 = plsc.load_gather(ee_v, [efj, _full16(c2)])
                ee1 = plsc.load_gather(ee_v, [efj, _full16(c2 + 1)])
                e0 = el0 + er0 + ee0
                e1 = el1 + er1 + ee1
                e0 = jnp.where(e0 > 0, e0, 0.2 * e0)
                e1 = jnp.where(e1 > 0, e1, 0.2 * e1)
                w0 = jnp.exp(e0)
                w1 = jnp.exp(e1)
                colw = plsc.load_gather(idx_v, [eidx, _full16(5)])
                plsc.store_scatter(plw_v, [eidx, colw], w0)
                plsc.store_scatter(plw_v, [eidx, colw + 1], w1)
                for f in range(16):
                    v0 = plsc.load_gather(sr_v, [eidx, _full16(f)])
                    plsc.store_scatter(pl_v, [eidx, _full16(f)], v0 * w0)
                    v1 = plsc.load_gather(sr_v, [eidx, _full16(16 + f)])
                    plsc.store_scatter(pl_v, [eidx, _full16(16 + f)],
                                       v1 * w1)
            pltpu.sync_copy(pl_v, accf_sh.at[dsts_v], add=True)
            pltpu.sync_copy(plw_v, accw_sh.at[dstw_v], add=True)
            for j in range(C // 16):
                eidx = ids16 + j * 16
                colw = plsc.load_gather(idx_v, [eidx, _full16(5)])
                plsc.store_scatter(plw_v, [eidx, colw], zero16)
                plsc.store_scatter(plw_v, [eidx, colw + 1], zero16)
            return 0

        lax.fori_loop(0, NCPT, chunk, 0)

        plsc.subcore_barrier()

        def drain(i, _):
            t = s + i * NSUB

            @pl.when(t < NDRAIN)
            def _():
                pltpu.sync_copy(accf_sh.at[pl.ds(t * DR, DR)],
                                outf_hbm.at[pl.ds(c * N + t * DR, DR)])

            @pl.when(t < ACCW // DR)
            def _():
                pltpu.sync_copy(accw_sh.at[pl.ds(t * DR, DR)],
                                outw_hbm.at[pl.ds(c * ACCW + t * DR, DR)])

            return 0

        lax.fori_loop(0, nzmax, drain, 0)

        @pl.when(s == NDRAIN % NSUB)
        def _():
            pltpu.sync_copy(accf_sh.at[pl.ds(NDRAIN * DR, DRAIN_REM)],
                            outf_hbm.at[pl.ds(c * N + NDRAIN * DR,
                                              DRAIN_REM)])

    return k(ta2, tb, idx, ee)


def _norm_body(acc_ref, w_ref, out_ref):
    parts = []
    for c in range(2):
        a = acc_ref[c]
        w = w_ref[c]
        parts.append(a[:, 0:16] / (w[:, 0:1] + 1e-9))
        parts.append(a[:, 16:32] / (w[:, 1:2] + 1e-9))
    out_ref[...] = jnp.concatenate(parts, axis=1)


def _normalize(acc, wpair):
    BN = 2000
    return pl.pallas_call(
        _norm_body,
        grid=(N // BN,),
        in_specs=[
            pl.BlockSpec((2, BN, ROWA), lambda i: (0, i, 0)),
            pl.BlockSpec((2, BN, 2), lambda i: (0, i, 0)),
        ],
        out_specs=pl.BlockSpec((BN, H * 16), lambda i: (i, 0)),
        out_shape=jax.ShapeDtypeStruct((N, H * 16), jnp.float32),
    )(acc, wpair)


def kernel(feat, edge_index, e_feat, fc, edge_emb, fc_e_w, attn_l, attn_r,
           attn_e):
    al8 = jnp.zeros((8, 16), jnp.float32).at[0:H].set(attn_l[0])
    ar8 = jnp.zeros((8, 16), jnp.float32).at[0:H].set(attn_r[0])
    ae8 = jnp.zeros((8, 16), jnp.float32).at[0:H].set(attn_e[0])
    emb8 = jnp.zeros((8, EF), jnp.float32).at[0:NET].set(edge_emb)

    ta, tb = _node_tables(feat, fc[0], fc[1], al8, ar8)
    ee = _ee_table(emb8, fc_e_w, ae8)

    src = edge_index[0]
    dst = edge_index[1]
    npad = EPAD - E
    zpad = jnp.zeros((npad,), jnp.int32)
    srcp = jnp.concatenate([src, zpad])
    dstc = jnp.concatenate([dst, zpad])
    dsts = jnp.concatenate([dst, jnp.full((npad,), N, jnp.int32)])
    efp = jnp.concatenate([e_feat, zpad])
    dstw = dsts >> 3
    colw = (dsts & 7) * 2
    zcol = jnp.zeros((EPAD,), jnp.int32)
    idx = jnp.stack(
        [srcp, dstc, efp, dsts, dstw, colw, zcol, zcol], axis=1)

    accf, accw = _sc_edge(ta.reshape(2 * N, ROWA), tb, idx, ee)

    wpair = accw.reshape(2, ACCW * ROWB)[:, :2 * N].reshape(2, N, 2)
    out = _normalize(accf.reshape(2, N, ROWA), wpair)
    return out.reshape(N, H, NTYPE * OUT_F)


# speedup vs baseline: 27.3467x; 1.1342x over previous
import functools

import jax
import jax.numpy as jnp
from jax import lax
from jax.experimental import pallas as pl
from jax.experimental.pallas import tpu as pltpu
from jax.experimental.pallas import tpu_sc as plsc

N = 50000
E = 800000
H = 4
IN_F = 64
OUT_F = 8
NTYPE = 2
EF = 16
NET = 5

ROWA = 32
ROWB = 16
C = 128
NSUB = 16
EPAD = ((E + C * NSUB - 1) // (C * NSUB)) * (C * NSUB)
NCHUNK = EPAD // C
NCPT = NCHUNK // NSUB
ACCR = N + 16
ACCW = 6272
DR = 128
NDRAIN = N // DR
DRAIN_REM = N - NDRAIN * DR


def _tables_body(feat_ref, fc0_ref, fc1_ref, al_ref, ar_ref, ta_ref, tb_ref):
    f = feat_ref[...]
    fs0 = jnp.dot(f[:, :IN_F], fc0_ref[...],
                  preferred_element_type=jnp.float32)
    fs1 = jnp.dot(f[:, IN_F:], fc1_ref[...],
                  preferred_element_type=jnp.float32)
    al = al_ref[...]
    ar = ar_ref[...]
    bn = f.shape[0]
    fsrc = []
    els = []
    ers = []
    for h in range(H):
        fh = jnp.concatenate([fs0[:, 8 * h:8 * h + 8],
                              fs1[:, 8 * h:8 * h + 8]], axis=1)
        fsrc.append(fh)
        els.append(jnp.sum(fh * al[h][None, :], axis=1, keepdims=True))
        ers.append(jnp.sum(fh * ar[h][None, :], axis=1, keepdims=True))
    zeros8 = jnp.zeros((bn, 8), jnp.float32)
    for c in range(2):
        ta_ref[c] = jnp.concatenate(
            [fsrc[2 * c], fsrc[2 * c + 1]], axis=1)
    tb_ref[...] = jnp.concatenate(els + ers + [zeros8], axis=1)


def _node_tables(feat, fc0, fc1, al8, ar8):
    BN = 2000
    grid = (N // BN,)
    return pl.pallas_call(
        _tables_body,
        grid=grid,
        in_specs=[
            pl.BlockSpec((BN, NTYPE * IN_F), lambda i: (i, 0)),
            pl.BlockSpec((IN_F, OUT_F * H), lambda i: (0, 0)),
            pl.BlockSpec((IN_F, OUT_F * H), lambda i: (0, 0)),
            pl.BlockSpec((8, 16), lambda i: (0, 0)),
            pl.BlockSpec((8, 16), lambda i: (0, 0)),
        ],
        out_specs=[
            pl.BlockSpec((2, BN, ROWA), lambda i: (0, i, 0)),
            pl.BlockSpec((BN, ROWB), lambda i: (i, 0)),
        ],
        out_shape=[
            jax.ShapeDtypeStruct((2, N, ROWA), jnp.float32),
            jax.ShapeDtypeStruct((N, ROWB), jnp.float32),
        ],
    )(feat, fc0, fc1, al8, ar8)


def _ee_body(emb_ref, w_ref, ae_ref, out_ref):
    eh = jnp.dot(emb_ref[...], w_ref[...],
                 preferred_element_type=jnp.float32)
    ae = ae_ref[...]
    cols = []
    for h in range(H):
        cols.append(jnp.sum(eh[:, 16 * h:16 * h + 16] * ae[h][None, :],
                            axis=1, keepdims=True))
    out_ref[...] = jnp.concatenate(cols, axis=1)


def _ee_table(emb8, fc_e_w, ae8):
    return pl.pallas_call(
        _ee_body,
        out_shape=jax.ShapeDtypeStruct((8, H), jnp.float32),
    )(emb8, fc_e_w, ae8)


def _full16(val):
    return jnp.full((16,), val, dtype=jnp.int32)


def _sc_edge(ta2, tb, idx, ee):
    mesh = plsc.VectorSubcoreMesh(core_axis_name="c", subcore_axis_name="s",
                                  num_cores=2, num_subcores=NSUB)

    @functools.partial(
        pl.kernel,
        mesh=mesh,
        compiler_params=pltpu.CompilerParams(
            needs_layout_passes=False, use_tc_tiling_on_sc=False),
        out_type=[
            jax.ShapeDtypeStruct((2 * N, ROWA), jnp.float32),
            jax.ShapeDtypeStruct((2 * ACCW, ROWB), jnp.float32),
        ],
        scratch_types=[
            pltpu.VMEM((C, 8), jnp.int32),
            pltpu.VMEM((C,), jnp.int32),
            pltpu.VMEM((C,), jnp.int32),
            pltpu.VMEM((C,), jnp.int32),
            pltpu.VMEM((C,), jnp.int32),
            pltpu.VMEM((C,), jnp.int32),
            pltpu.VMEM((C, ROWA), jnp.float32),
            pltpu.VMEM((C, ROWB), jnp.float32),
            pltpu.VMEM((C, ROWB), jnp.float32),
            pltpu.VMEM((C, ROWA), jnp.float32),
            pltpu.VMEM((C, ROWB), jnp.float32),
            pltpu.VMEM((8, H), jnp.float32),
            pltpu.VMEM_SHARED((ACCR, ROWA), jnp.float32),
            pltpu.VMEM_SHARED((ACCW, ROWB), jnp.float32),
            pltpu.SemaphoreType.DMA,
        ],
    )
    def k(ta_hbm, tb_hbm, idx_hbm, ee_hbm, outf_hbm, outw_hbm,
          idx_v, sadj_v, src_v, dstc_v, dsts_v, dstw_v,
          sr_v, lb_v, rb_v, pl_v, plw_v, ee_v, accf_sh, accw_sh, sem1):
        c = lax.axis_index("c")
        s = lax.axis_index("s")
        ids16 = lax.iota(jnp.int32, 16)
        zero16 = jnp.zeros((16,), jnp.float32)
        cN16 = _full16(c * N)
        c2 = 2 * c

        for j in range(C // 16):
            eidx = ids16 + j * 16
            for col in range(ROWA):
                plsc.store_scatter(pl_v, [eidx, _full16(col)], zero16)
            for col in range(ROWB):
                plsc.store_scatter(plw_v, [eidx, _full16(col)], zero16)

        pltpu.sync_copy(ee_hbm, ee_v)

        def zacc(i, _):
            t = s + i * NSUB

            @pl.when(t < NDRAIN)
            def _():
                pltpu.sync_copy(pl_v, accf_sh.at[pl.ds(t * DR, DR)])

            @pl.when(t < ACCW // DR)
            def _():
                pltpu.sync_copy(plw_v, accw_sh.at[pl.ds(t * DR, DR)])

            return 0

        nzmax = (NDRAIN + NSUB - 1) // NSUB
        lax.fori_loop(0, nzmax, zacc, 0)

        @pl.when(s == NDRAIN % NSUB)
        def _():
            pltpu.sync_copy(pl_v.at[pl.ds(0, DRAIN_REM + 16)],
                            accf_sh.at[pl.ds(NDRAIN * DR, DRAIN_REM + 16)])

        plsc.subcore_barrier()

        def chunk(i, _):
            base = (s + i * NSUB) * C
            pltpu.sync_copy(idx_hbm.at[pl.ds(base, C)], idx_v)
            for j in range(C // 16):
                eidx = ids16 + j * 16
                g0 = plsc.load_gather(idx_v, [eidx, _full16(0)])
                plsc.store_scatter(src_v, [eidx], g0)
                plsc.store_scatter(sadj_v, [eidx], g0 + cN16)
                g1 = plsc.load_gather(idx_v, [eidx, _full16(1)])
                plsc.store_scatter(dstc_v, [eidx], g1)
                g3 = plsc.load_gather(idx_v, [eidx, _full16(3)])
                plsc.store_scatter(dsts_v, [eidx], g3)
                g4 = plsc.load_gather(idx_v, [eidx, _full16(4)])
                plsc.store_scatter(dstw_v, [eidx], g4)
            d1 = pltpu.async_copy(ta_hbm.at[sadj_v], sr_v, sem1)
            d2 = pltpu.async_copy(tb_hbm.at[src_v], lb_v, sem1)
            d3 = pltpu.async_copy(tb_hbm.at[dstc_v], rb_v, sem1)
            d1.wait()
            d2.wait()
            d3.wait()
            for j in range(C // 16):
                eidx = ids16 + j * 16
                el0 = plsc.load_gather(lb_v, [eidx, _full16(c2)])
                el1 = plsc.load_gather(lb_v, [eidx, _full16(c2 + 1)])
                er0 = plsc.load_gather(rb_v, [eidx, _full16(4 + c2)])
                er1 = plsc.load_gather(rb_v, [eidx, _full16(4 + c2 + 1)])
                efj = plsc.load_gather(idx_v, [eidx, _full16(2)])
                ee0 = plsc.load_gather(ee_v, [efj, _full16(c2)])
                ee1 = plsc.load_gather(ee_v, [efj, _full16(c2 + 1)])
                e0 = el0 + er0 + ee0
                e1 = el1 + er1 + ee1
                e0 = jnp.where(e0 > 0, e0, 0.2 * e0)
                e1 = jnp.where(e1 > 0, e1, 0.2 * e1)
                w0 = jnp.exp(e0)
                w1 = jnp.exp(e1)
                colw = plsc.load_gather(idx_v, [eidx, _full16(5)])
                plsc.store_scatter(plw_v, [eidx, colw], w0)
                plsc.store_scatter(plw_v, [eidx, colw + 1], w1)
                for f in range(16):
                    v0 = plsc.load_gather(sr_v, [eidx, _full16(f)])
                    plsc.store_scatter(pl_v, [eidx, _full16(f)], v0 * w0)
                    v1 = plsc.load_gather(sr_v, [eidx, _full16(16 + f)])
                    plsc.store_scatter(pl_v, [eidx, _full16(16 + f)],
                                       v1 * w1)
            s1 = pltpu.async_copy(pl_v, accf_sh.at[dsts_v], sem1, add=True)
            s2 = pltpu.async_copy(plw_v, accw_sh.at[dstw_v], sem1, add=True)
            s1.wait()
            s2.wait()
            for j in range(C // 16):
                eidx = ids16 + j * 16
                colw = plsc.load_gather(idx_v, [eidx, _full16(5)])
                plsc.store_scatter(plw_v, [eidx, colw], zero16)
                plsc.store_scatter(plw_v, [eidx, colw + 1], zero16)
            return 0

        lax.fori_loop(0, NCPT, chunk, 0)

        plsc.subcore_barrier()

        def drain(i, _):
            t = s + i * NSUB

            @pl.when(t < NDRAIN)
            def _():
                pltpu.sync_copy(accf_sh.at[pl.ds(t * DR, DR)],
                                outf_hbm.at[pl.ds(c * N + t * DR, DR)])

            @pl.when(t < ACCW // DR)
            def _():
                pltpu.sync_copy(accw_sh.at[pl.ds(t * DR, DR)],
                                outw_hbm.at[pl.ds(c * ACCW + t * DR, DR)])

            return 0

        lax.fori_loop(0, nzmax, drain, 0)

        @pl.when(s == NDRAIN % NSUB)
        def _():
            pltpu.sync_copy(accf_sh.at[pl.ds(NDRAIN * DR, DRAIN_REM)],
                            outf_hbm.at[pl.ds(c * N + NDRAIN * DR,
                                              DRAIN_REM)])

    return k(ta2, tb, idx, ee)


def _norm_body(acc_ref, w_ref, out_ref):
    parts = []
    for c in range(2):
        a = acc_ref[c]
        w = w_ref[c]
        parts.append(a[:, 0:16] / (w[:, 0:1] + 1e-9))
        parts.append(a[:, 16:32] / (w[:, 1:2] + 1e-9))
    out_ref[...] = jnp.concatenate(parts, axis=1)


def _normalize(acc, wpair):
    BN = 2000
    return pl.pallas_call(
        _norm_body,
        grid=(N // BN,),
        in_specs=[
            pl.BlockSpec((2, BN, ROWA), lambda i: (0, i, 0)),
            pl.BlockSpec((2, BN, 2), lambda i: (0, i, 0)),
        ],
        out_specs=pl.BlockSpec((BN, H * 16), lambda i: (i, 0)),
        out_shape=jax.ShapeDtypeStruct((N, H * 16), jnp.float32),
    )(acc, wpair)


def kernel(feat, edge_index, e_feat, fc, edge_emb, fc_e_w, attn_l, attn_r,
           attn_e):
    al8 = jnp.zeros((8, 16), jnp.float32).at[0:H].set(attn_l[0])
    ar8 = jnp.zeros((8, 16), jnp.float32).at[0:H].set(attn_r[0])
    ae8 = jnp.zeros((8, 16), jnp.float32).at[0:H].set(attn_e[0])
    emb8 = jnp.zeros((8, EF), jnp.float32).at[0:NET].set(edge_emb)

    ta, tb = _node_tables(feat, fc[0], fc[1], al8, ar8)
    ee = _ee_table(emb8, fc_e_w, ae8)

    src = edge_index[0]
    dst = edge_index[1]
    npad = EPAD - E
    zpad = jnp.zeros((npad,), jnp.int32)
    srcp = jnp.concatenate([src, zpad])
    dstc = jnp.concatenate([dst, zpad])
    dsts = jnp.concatenate([dst, jnp.full((npad,), N, jnp.int32)])
    efp = jnp.concatenate([e_feat, zpad])
    dstw = dsts >> 3
    colw = (dsts & 7) * 2
    zcol = jnp.zeros((EPAD,), jnp.int32)
    idx = jnp.stack(
        [srcp, dstc, efp, dsts, dstw, colw, zcol, zcol], axis=1)

    accf, accw = _sc_edge(ta.reshape(2 * N, ROWA), tb, idx, ee)

    wpair = accw.reshape(2, ACCW * ROWB)[:, :2 * N].reshape(2, N, 2)
    out = _normalize(accf.reshape(2, N, ROWA), wpair)
    return out.reshape(N, H, NTYPE * OUT_F)
